# Initial kernel scaffold; baseline (speedup 1.0000x reference)
import jax
import jax.numpy as jnp
from jax.experimental import pallas as pl


def kernel(x, W_patch, W_i2c1, b_i2c1, W_i2c2, b_i2c2, W_proj, b_proj, class_emb, W_g1, b_g1, W_g2, b_g2, edge_index):
    raise NotImplementedError("write your pallas kernel here")


# speedup vs baseline: 2.5097x; 2.5097x over previous
import jax
import jax.numpy as jnp
from jax import lax
from jax.experimental import pallas as pl

NC = 26
ER = 7
B = 64
P = 16
NP = 14


def _mean_kernel(x_ref, o_ref):
    xr = x_ref[0]
    s = xr[:, 0:P, :]
    for ph in range(1, NP):
        s = s + xr[:, ph * P:(ph + 1) * P, :]
    s = s.reshape(3 * P, NP * P)
    w_id = lax.broadcasted_iota(jnp.int32, (NP * P, P), 0)
    j_id = lax.broadcasted_iota(jnp.int32, (NP * P, P), 1)
    comb = (w_id % P == j_id).astype(jnp.float32)
    out = jnp.dot(s, comb, preferred_element_type=jnp.float32)
    o_ref[0] = out * (1.0 / (NP * NP))


def _head_kernel(m_ref, wpp_ref, w1_ref, b1_ref, w2_ref, b2_ref,
                 wp_ref, bp_ref, emb_ref, wg1_ref, bg1_ref,
                 wg2_ref, bg2_ref, src_ref, dst_ref, o_ref):
    f32 = jnp.float32
    m = m_ref[...]
    feat = jnp.dot(m, wpp_ref[...], preferred_element_type=f32)

    t1 = jnp.maximum(jnp.dot(feat, w1_ref[...], preferred_element_type=f32)
                     + b1_ref[0], 0.0)
    cnn_logits = jnp.dot(t1, w2_ref[...], preferred_element_type=f32) + b2_ref[0]

    proj = jnp.maximum(jnp.dot(feat, wp_ref[...], preferred_element_type=f32)
                       + bp_ref[0], 0.0)

    src = src_ref[...]
    dst = dst_ref[...]
    node = lax.broadcasted_iota(jnp.int32, (ER, NC, NC), 2)
    oh_dst = (dst[:, :, None] == node).astype(f32)
    oh_src = (src[:, :, None] == node).astype(f32)
    deg = jnp.sum(oh_dst, axis=(0, 1)) + 1.0
    dinv = lax.rsqrt(deg)
    dinv_s = jnp.sum(oh_src * dinv[None, None, :], axis=2)
    dinv_d = jnp.sum(oh_dst * dinv[None, None, :], axis=2)
    norm = dinv_s * dinv_d
    lhs = (oh_dst * norm[:, :, None]).reshape(ER * NC, NC)
    rhs = oh_src.reshape(ER * NC, NC)
    adj = lax.dot_general(lhs, rhs, (((0,), (0,)), ((), ())),
                          preferred_element_type=f32)
    r_id = lax.broadcasted_iota(jnp.int32, (NC, NC), 0)
    c_id = lax.broadcasted_iota(jnp.int32, (NC, NC), 1)
    adj = adj + jnp.where(r_id == c_id, dinv * dinv, 0.0)[:, :]

    g1 = jnp.dot(emb_ref[...], wg1_ref[...], preferred_element_type=f32)
    h1 = jnp.maximum(jnp.dot(adj, g1, preferred_element_type=f32) + bg1_ref[0], 0.0)
    g2 = jnp.dot(h1, wg2_ref[...], preferred_element_type=f32)
    h2 = jnp.dot(adj, g2, preferred_element_type=f32) + bg2_ref[0]

    gcn_logits = lax.dot_general(proj, h2, (((1,), (1,)), ((), ())),
                                 preferred_element_type=f32)
    o_ref[...] = cnn_logits + gcn_logits


def kernel(x, W_patch, W_i2c1, b_i2c1, W_i2c2, b_i2c2, W_proj, b_proj,
           class_emb, W_g1, b_g1, W_g2, b_g2, edge_index):
    W_pp = W_patch.reshape(P, P, 3, 2048).transpose(2, 0, 1, 3).reshape(768, 2048)
    src8 = edge_index[0].astype(jnp.int32).reshape(ER, NC)
    dst8 = edge_index[1].astype(jnp.int32).reshape(ER, NC)

    m3 = pl.pallas_call(
        _mean_kernel,
        grid=(B,),
        in_specs=[pl.BlockSpec((1, 3, NP * P, NP * P), lambda b: (b, 0, 0, 0))],
        out_specs=pl.BlockSpec((1, 3 * P, P), lambda b: (b, 0, 0)),
        out_shape=jax.ShapeDtypeStruct((B, 3 * P, P), jnp.float32),
    )(x)
    m = m3.reshape(B, 768)

    out = pl.pallas_call(
        _head_kernel,
        out_shape=jax.ShapeDtypeStruct((B, NC), jnp.float32),
    )(m, W_pp,
      W_i2c1, b_i2c1.reshape(1, -1), W_i2c2, b_i2c2.reshape(1, -1),
      W_proj, b_proj.reshape(1, -1), class_emb,
      W_g1, b_g1.reshape(1, -1), W_g2, b_g2.reshape(1, -1),
      src8, dst8)
    return out


# speedup vs baseline: 3.5269x; 1.4053x over previous
import jax
import jax.numpy as jnp
from jax import lax
from jax.experimental import pallas as pl
from jax.experimental.pallas import tpu as pltpu

NC = 26
ER = 7
B = 64
P = 16
NP = 14


def _fused_kernel(x_ref, wpat_ref, w1_ref, b1_ref, w2_ref, b2_ref,
                  wp_ref, bp_ref, emb_ref, wg1_ref, bg1_ref,
                  wg2_ref, bg2_ref, src_ref, dst_ref, o_ref, m_ref):
    f32 = jnp.float32
    b = pl.program_id(0)

    xr = x_ref[0]
    s = xr[:, 0:P, :]
    for ph in range(1, NP):
        s = s + xr[:, ph * P:(ph + 1) * P, :]
    w_id = lax.broadcasted_iota(jnp.int32, (NP * P, 3 * P), 0)
    k_id = lax.broadcasted_iota(jnp.int32, (NP * P, 3 * P), 1)
    row = jnp.zeros((P, 3 * P), f32)
    for c in range(3):
        comb_c = ((k_id % 3 == c) & (w_id % P == k_id // 3)).astype(f32)
        row = row + jnp.dot(s[c], comb_c, preferred_element_type=f32)
    flat = jnp.concatenate([row[i:i + 1, :] for i in range(P)], axis=1)
    m_ref[pl.ds(b, 1), :] = flat * (1.0 / (NP * NP))

    @pl.when(b == B - 1)
    def _head():
        m = m_ref[...]
        feat = jnp.dot(m, wpat_ref[...], preferred_element_type=f32)

        t1 = jnp.maximum(jnp.dot(feat, w1_ref[...], preferred_element_type=f32)
                         + b1_ref[0], 0.0)
        cnn_logits = jnp.dot(t1, w2_ref[...], preferred_element_type=f32) + b2_ref[0]

        proj = jnp.maximum(jnp.dot(feat, wp_ref[...], preferred_element_type=f32)
                           + bp_ref[0], 0.0)

        src = src_ref[...]
        dst = dst_ref[...]
        node = lax.broadcasted_iota(jnp.int32, (ER, NC, NC), 2)
        oh_dst = (dst[:, :, None] == node).astype(f32)
        oh_src = (src[:, :, None] == node).astype(f32)
        deg = jnp.sum(oh_dst, axis=(0, 1)) + 1.0
        dinv = lax.rsqrt(deg)
        dinv_s = jnp.sum(oh_src * dinv[None, None, :], axis=2)
        dinv_d = jnp.sum(oh_dst * dinv[None, None, :], axis=2)
        norm = dinv_s * dinv_d
        lhs = (oh_dst * norm[:, :, None]).reshape(ER * NC, NC)
        rhs = oh_src.reshape(ER * NC, NC)
        adj = lax.dot_general(lhs, rhs, (((0,), (0,)), ((), ())),
                              preferred_element_type=f32)
        r_id = lax.broadcasted_iota(jnp.int32, (NC, NC), 0)
        c_id = lax.broadcasted_iota(jnp.int32, (NC, NC), 1)
        adj = adj + jnp.where(r_id == c_id, dinv * dinv, 0.0)

        g1 = jnp.dot(emb_ref[...], wg1_ref[...], preferred_element_type=f32)
        h1 = jnp.maximum(jnp.dot(adj, g1, preferred_element_type=f32)
                         + bg1_ref[0], 0.0)
        g2 = jnp.dot(h1, wg2_ref[...], preferred_element_type=f32)
        h2 = jnp.dot(adj, g2, preferred_element_type=f32) + bg2_ref[0]

        gcn_logits = lax.dot_general(proj, h2, (((1,), (1,)), ((), ())),
                                     preferred_element_type=f32)
        o_ref[...] = cnn_logits + gcn_logits


def kernel(x, W_patch, W_i2c1, b_i2c1, W_i2c2, b_i2c2, W_proj, b_proj,
           class_emb, W_g1, b_g1, W_g2, b_g2, edge_index):
    src8 = edge_index[0].astype(jnp.int32).reshape(ER, NC)
    dst8 = edge_index[1].astype(jnp.int32).reshape(ER, NC)

    def const(shape):
        n = len(shape)
        return pl.BlockSpec(shape, lambda b, _n=n: (0,) * _n)

    out = pl.pallas_call(
        _fused_kernel,
        grid=(B,),
        in_specs=[
            pl.BlockSpec((1, 3, NP * P, NP * P), lambda b: (b, 0, 0, 0)),
            const((768, 2048)),
            const((2048, 1024)), const((1, 1024)),
            const((1024, NC)), const((1, NC)),
            const((2048, 1024)), const((1, 1024)),
            const((NC, 1024)),
            const((1024, 512)), const((1, 512)),
            const((512, 1024)), const((1, 1024)),
            const((ER, NC)), const((ER, NC)),
        ],
        out_specs=const((B, NC)),
        out_shape=jax.ShapeDtypeStruct((B, NC), jnp.float32),
        scratch_shapes=[pltpu.VMEM((B, 3 * P * P), jnp.float32)],
    )(x, W_patch,
      W_i2c1, b_i2c1.reshape(1, -1), W_i2c2, b_i2c2.reshape(1, -1),
      W_proj, b_proj.reshape(1, -1), class_emb,
      W_g1, b_g1.reshape(1, -1), W_g2, b_g2.reshape(1, -1),
      src8, dst8)
    return out


# speedup vs baseline: 6.2827x; 1.7814x over previous
import jax
import jax.numpy as jnp
from jax import lax
from jax.experimental import pallas as pl
from jax.experimental.pallas import tpu as pltpu

NC = 26
ER = 7
B = 64
BS = 4
P = 16
NP = 14


def _fused_kernel(x_ref, wpat_ref, w1_ref, b1_ref, w2_ref, b2_ref,
                  wp_ref, bp_ref, emb_ref, wg1_ref, bg1_ref,
                  wg2_ref, bg2_ref, src_ref, dst_ref, o_ref, m_ref):
    f32 = jnp.float32
    b = pl.program_id(0)

    w_id = lax.broadcasted_iota(jnp.int32, (NP * P, 3 * P), 0)
    k_id = lax.broadcasted_iota(jnp.int32, (NP * P, 3 * P), 1)
    combs = [((k_id % 3 == c) & (w_id % P == k_id // 3)).astype(f32)
             for c in range(3)]

    for bi in range(BS):
        xr = x_ref[bi]
        s = xr[:, 0:P, :]
        for ph in range(1, NP):
            s = s + xr[:, ph * P:(ph + 1) * P, :]
        row = jnp.zeros((P, 3 * P), f32)
        for c in range(3):
            row = row + jnp.dot(s[c], combs[c], preferred_element_type=f32)
        flat = jnp.concatenate([row[i:i + 1, :] for i in range(P)], axis=1)
        m_ref[pl.ds(b * BS + bi, 1), :] = flat * (1.0 / (NP * NP))

    @pl.when(b == B // BS - 1)
    def _head():
        m = m_ref[...]
        feat = jnp.dot(m, wpat_ref[...], preferred_element_type=f32)

        t1 = jnp.maximum(jnp.dot(feat, w1_ref[...], preferred_element_type=f32)
                         + b1_ref[0], 0.0)
        cnn_logits = jnp.dot(t1, w2_ref[...], preferred_element_type=f32) + b2_ref[0]

        proj = jnp.maximum(jnp.dot(feat, wp_ref[...], preferred_element_type=f32)
                           + bp_ref[0], 0.0)

        src = src_ref[...]
        dst = dst_ref[...]
        node = lax.broadcasted_iota(jnp.int32, (ER, NC, NC), 2)
        oh_dst = (dst[:, :, None] == node).astype(f32)
        oh_src = (src[:, :, None] == node).astype(f32)
        deg = jnp.sum(oh_dst, axis=(0, 1)) + 1.0
        dinv = lax.rsqrt(deg)
        dinv_s = jnp.sum(oh_src * dinv[None, None, :], axis=2)
        dinv_d = jnp.sum(oh_dst * dinv[None, None, :], axis=2)
        norm = dinv_s * dinv_d
        lhs = (oh_dst * norm[:, :, None]).reshape(ER * NC, NC)
        rhs = oh_src.reshape(ER * NC, NC)
        adj = lax.dot_general(lhs, rhs, (((0,), (0,)), ((), ())),
                              preferred_element_type=f32)
        r_id = lax.broadcasted_iota(jnp.int32, (NC, NC), 0)
        c_id = lax.broadcasted_iota(jnp.int32, (NC, NC), 1)
        adj = adj + jnp.where(r_id == c_id, dinv * dinv, 0.0)

        g1 = jnp.dot(emb_ref[...], wg1_ref[...], preferred_element_type=f32)
        h1 = jnp.maximum(jnp.dot(adj, g1, preferred_element_type=f32)
                         + bg1_ref[0], 0.0)
        g2 = jnp.dot(h1, wg2_ref[...], preferred_element_type=f32)
        h2 = jnp.dot(adj, g2, preferred_element_type=f32) + bg2_ref[0]

        gcn_logits = lax.dot_general(proj, h2, (((1,), (1,)), ((), ())),
                                     preferred_element_type=f32)
        o_ref[...] = cnn_logits + gcn_logits


def kernel(x, W_patch, W_i2c1, b_i2c1, W_i2c2, b_i2c2, W_proj, b_proj,
           class_emb, W_g1, b_g1, W_g2, b_g2, edge_index):
    src8 = edge_index[0].astype(jnp.int32).reshape(ER, NC)
    dst8 = edge_index[1].astype(jnp.int32).reshape(ER, NC)

    def const(shape):
        n = len(shape)
        return pl.BlockSpec(shape, lambda b, _n=n: (0,) * _n)

    out = pl.pallas_call(
        _fused_kernel,
        grid=(B // BS,),
        in_specs=[
            pl.BlockSpec((BS, 3, NP * P, NP * P), lambda b: (b, 0, 0, 0)),
            const((768, 2048)),
            const((2048, 1024)), const((1, 1024)),
            const((1024, NC)), const((1, NC)),
            const((2048, 1024)), const((1, 1024)),
            const((NC, 1024)),
            const((1024, 512)), const((1, 512)),
            const((512, 1024)), const((1, 1024)),
            const((ER, NC)), const((ER, NC)),
        ],
        out_specs=const((B, NC)),
        out_shape=jax.ShapeDtypeStruct((B, NC), jnp.float32),
        scratch_shapes=[pltpu.VMEM((B, 3 * P * P), jnp.float32)],
    )(x, W_patch,
      W_i2c1, b_i2c1.reshape(1, -1), W_i2c2, b_i2c2.reshape(1, -1),
      W_proj, b_proj.reshape(1, -1), class_emb,
      W_g1, b_g1.reshape(1, -1), W_g2, b_g2.reshape(1, -1),
      src8, dst8)
    return out


# speedup vs baseline: 7.1969x; 1.1455x over previous
import jax
import jax.numpy as jnp
from jax import lax
from jax.experimental import pallas as pl
from jax.experimental.pallas import tpu as pltpu

NC = 26
ER = 7
B = 64
BS = 8
P = 16
NP = 14


def _fused_kernel(x_ref, wpat_ref, w1_ref, b1_ref, w2_ref, b2_ref,
                  wp_ref, bp_ref, emb_ref, wg1_ref, bg1_ref,
                  wg2_ref, bg2_ref, src_ref, dst_ref, o_ref, m_ref):
    f32 = jnp.float32
    b = pl.program_id(0)

    w_id = lax.broadcasted_iota(jnp.int32, (NP * P, 3 * P), 0)
    k_id = lax.broadcasted_iota(jnp.int32, (NP * P, 3 * P), 1)
    combs = [((k_id % 3 == c) & (w_id % P == k_id // 3)).astype(f32)
             for c in range(3)]

    for bi in range(BS):
        xr = x_ref[bi]
        s = xr[:, 0:P, :]
        for ph in range(1, NP):
            s = s + xr[:, ph * P:(ph + 1) * P, :]
        row = jnp.zeros((P, 3 * P), f32)
        for c in range(3):
            row = row + jnp.dot(s[c], combs[c], preferred_element_type=f32)
        flat = jnp.concatenate([row[i:i + 1, :] for i in range(P)], axis=1)
        m_ref[pl.ds(b * BS + bi, 1), :] = flat * (1.0 / (NP * NP))

    @pl.when(b == B // BS - 1)
    def _head():
        m = m_ref[...]
        feat = jnp.dot(m, wpat_ref[...], preferred_element_type=f32)

        t1 = jnp.maximum(jnp.dot(feat, w1_ref[...], preferred_element_type=f32)
                         + b1_ref[0], 0.0)
        cnn_logits = jnp.dot(t1, w2_ref[...], preferred_element_type=f32) + b2_ref[0]

        proj = jnp.maximum(jnp.dot(feat, wp_ref[...], preferred_element_type=f32)
                           + bp_ref[0], 0.0)

        src = src_ref[...]
        dst = dst_ref[...]
        node = lax.broadcasted_iota(jnp.int32, (ER, NC, NC), 2)
        oh_dst = (dst[:, :, None] == node).astype(f32)
        oh_src = (src[:, :, None] == node).astype(f32)
        deg = jnp.sum(oh_dst, axis=(0, 1)) + 1.0
        dinv = lax.rsqrt(deg)
        dinv_s = jnp.sum(oh_src * dinv[None, None, :], axis=2)
        dinv_d = jnp.sum(oh_dst * dinv[None, None, :], axis=2)
        norm = dinv_s * dinv_d
        lhs = (oh_dst * norm[:, :, None]).reshape(ER * NC, NC)
        rhs = oh_src.reshape(ER * NC, NC)
        adj = lax.dot_general(lhs, rhs, (((0,), (0,)), ((), ())),
                              preferred_element_type=f32)
        r_id = lax.broadcasted_iota(jnp.int32, (NC, NC), 0)
        c_id = lax.broadcasted_iota(jnp.int32, (NC, NC), 1)
        adj = adj + jnp.where(r_id == c_id, dinv * dinv, 0.0)

        g1 = jnp.dot(emb_ref[...], wg1_ref[...], preferred_element_type=f32)
        h1 = jnp.maximum(jnp.dot(adj, g1, preferred_element_type=f32)
                         + bg1_ref[0], 0.0)
        g2 = jnp.dot(h1, wg2_ref[...], preferred_element_type=f32)
        h2 = jnp.dot(adj, g2, preferred_element_type=f32) + bg2_ref[0]

        gcn_logits = lax.dot_general(proj, h2, (((1,), (1,)), ((), ())),
                                     preferred_element_type=f32)
        o_ref[...] = cnn_logits + gcn_logits


def kernel(x, W_patch, W_i2c1, b_i2c1, W_i2c2, b_i2c2, W_proj, b_proj,
           class_emb, W_g1, b_g1, W_g2, b_g2, edge_index):
    src8 = edge_index[0].astype(jnp.int32).reshape(ER, NC)
    dst8 = edge_index[1].astype(jnp.int32).reshape(ER, NC)

    def const(shape):
        n = len(shape)
        return pl.BlockSpec(shape, lambda b, _n=n: (0,) * _n)

    out = pl.pallas_call(
        _fused_kernel,
        grid=(B // BS,),
        in_specs=[
            pl.BlockSpec((BS, 3, NP * P, NP * P), lambda b: (b, 0, 0, 0)),
            const((768, 2048)),
            const((2048, 1024)), const((1, 1024)),
            const((1024, NC)), const((1, NC)),
            const((2048, 1024)), const((1, 1024)),
            const((NC, 1024)),
            const((1024, 512)), const((1, 512)),
            const((512, 1024)), const((1, 1024)),
            const((ER, NC)), const((ER, NC)),
        ],
        out_specs=const((B, NC)),
        out_shape=jax.ShapeDtypeStruct((B, NC), jnp.float32),
        scratch_shapes=[pltpu.VMEM((B, 3 * P * P), jnp.float32)],
    )(x, W_patch,
      W_i2c1, b_i2c1.reshape(1, -1), W_i2c2, b_i2c2.reshape(1, -1),
      W_proj, b_proj.reshape(1, -1), class_emb,
      W_g1, b_g1.reshape(1, -1), W_g2, b_g2.reshape(1, -1),
      src8, dst8)
    return out


# speedup vs baseline: 7.2377x; 1.0057x over previous
import jax
import jax.numpy as jnp
from jax import lax
from jax.experimental import pallas as pl
from jax.experimental.pallas import tpu as pltpu

NC = 26
ER = 7
B = 64
BS = 16
P = 16
NP = 14


def _fused_kernel(x_ref, wpat_ref, w1_ref, b1_ref, w2_ref, b2_ref,
                  wp_ref, bp_ref, emb_ref, wg1_ref, bg1_ref,
                  wg2_ref, bg2_ref, src_ref, dst_ref, o_ref, m_ref):
    f32 = jnp.float32
    b = pl.program_id(0)

    w_id = lax.broadcasted_iota(jnp.int32, (NP * P, 3 * P), 0)
    k_id = lax.broadcasted_iota(jnp.int32, (NP * P, 3 * P), 1)
    combs = [((k_id % 3 == c) & (w_id % P == k_id // 3)).astype(f32)
             for c in range(3)]

    for bi in range(BS):
        xr = x_ref[bi]
        s = xr[:, 0:P, :]
        for ph in range(1, NP):
            s = s + xr[:, ph * P:(ph + 1) * P, :]
        row = jnp.zeros((P, 3 * P), f32)
        for c in range(3):
            row = row + jnp.dot(s[c], combs[c], preferred_element_type=f32)
        flat = jnp.concatenate([row[i:i + 1, :] for i in range(P)], axis=1)
        m_ref[pl.ds(b * BS + bi, 1), :] = flat * (1.0 / (NP * NP))

    @pl.when(b == B // BS - 1)
    def _head():
        m = m_ref[...]
        feat = jnp.dot(m, wpat_ref[...], preferred_element_type=f32)

        t1 = jnp.maximum(jnp.dot(feat, w1_ref[...], preferred_element_type=f32)
                         + b1_ref[0], 0.0)
        cnn_logits = jnp.dot(t1, w2_ref[...], preferred_element_type=f32) + b2_ref[0]

        proj = jnp.maximum(jnp.dot(feat, wp_ref[...], preferred_element_type=f32)
                           + bp_ref[0], 0.0)

        src = src_ref[...]
        dst = dst_ref[...]
        node = lax.broadcasted_iota(jnp.int32, (ER, NC, NC), 2)
        oh_dst = (dst[:, :, None] == node).astype(f32)
        oh_src = (src[:, :, None] == node).astype(f32)
        deg = jnp.sum(oh_dst, axis=(0, 1)) + 1.0
        dinv = lax.rsqrt(deg)
        dinv_s = jnp.sum(oh_src * dinv[None, None, :], axis=2)
        dinv_d = jnp.sum(oh_dst * dinv[None, None, :], axis=2)
        norm = dinv_s * dinv_d
        lhs = (oh_dst * norm[:, :, None]).reshape(ER * NC, NC)
        rhs = oh_src.reshape(ER * NC, NC)
        adj = lax.dot_general(lhs, rhs, (((0,), (0,)), ((), ())),
                              preferred_element_type=f32)
        r_id = lax.broadcasted_iota(jnp.int32, (NC, NC), 0)
        c_id = lax.broadcasted_iota(jnp.int32, (NC, NC), 1)
        adj = adj + jnp.where(r_id == c_id, dinv * dinv, 0.0)

        g1 = jnp.dot(emb_ref[...], wg1_ref[...], preferred_element_type=f32)
        h1 = jnp.maximum(jnp.dot(adj, g1, preferred_element_type=f32)
                         + bg1_ref[0], 0.0)
        g2 = jnp.dot(h1, wg2_ref[...], preferred_element_type=f32)
        h2 = jnp.dot(adj, g2, preferred_element_type=f32) + bg2_ref[0]

        gcn_logits = lax.dot_general(proj, h2, (((1,), (1,)), ((), ())),
                                     preferred_element_type=f32)
        o_ref[...] = cnn_logits + gcn_logits


def kernel(x, W_patch, W_i2c1, b_i2c1, W_i2c2, b_i2c2, W_proj, b_proj,
           class_emb, W_g1, b_g1, W_g2, b_g2, edge_index):
    src8 = edge_index[0].astype(jnp.int32).reshape(ER, NC)
    dst8 = edge_index[1].astype(jnp.int32).reshape(ER, NC)

    def const(shape):
        n = len(shape)
        return pl.BlockSpec(shape, lambda b, _n=n: (0,) * _n)

    out = pl.pallas_call(
        _fused_kernel,
        grid=(B // BS,),
        in_specs=[
            pl.BlockSpec((BS, 3, NP * P, NP * P), lambda b: (b, 0, 0, 0)),
            const((768, 2048)),
            const((2048, 1024)), const((1, 1024)),
            const((1024, NC)), const((1, NC)),
            const((2048, 1024)), const((1, 1024)),
            const((NC, 1024)),
            const((1024, 512)), const((1, 512)),
            const((512, 1024)), const((1, 1024)),
            const((ER, NC)), const((ER, NC)),
        ],
        out_specs=const((B, NC)),
        out_shape=jax.ShapeDtypeStruct((B, NC), jnp.float32),
        scratch_shapes=[pltpu.VMEM((B, 3 * P * P), jnp.float32)],
    )(x, W_patch,
      W_i2c1, b_i2c1.reshape(1, -1), W_i2c2, b_i2c2.reshape(1, -1),
      W_proj, b_proj.reshape(1, -1), class_emb,
      W_g1, b_g1.reshape(1, -1), W_g2, b_g2.reshape(1, -1),
      src8, dst8)
    return out
